# baseline (speedup 1.0000x reference)
import functools

import jax
import jax.numpy as jnp
from jax import lax
from jax.experimental import pallas as pl
from jax.experimental.pallas import tpu as pltpu
from jax.experimental.pallas import tpu_sc as plsc

_NUM_FIELDS = 26
_FIELD_WIDTH = 38462
_EMBED_DIM = 32
_BATCH = 16384

_LANES = 16
_NW = 32
_NG = _BATCH // 128
_WINA = 38400
_WINB = 256
_WIN = _WINA + _WINB
_TAIL_START = 999936


def _body(xt_hbm, tab_hbm, tail_hbm, out_hbm, win_v, idx_v, outb_v, *sems):
    wsem = sems[0:2]
    isem = sems[2:4]
    osem = sems[4:6]
    wid = lax.axis_index("s") * 2 + lax.axis_index("c")
    cg = wid // 8
    ci = lax.rem(wid, 8)

    def fire_win(f, b):
        off = f * _FIELD_WIDTH
        d = lax.rem(off, 128)
        start = pl.multiple_of(off - d, 128)
        last = f == _NUM_FIELDS - 1
        for c in range(8):
            @pl.when(ci == c)
            def _(c=c):
                pltpu.async_copy(tab_hbm.at[cg, c, pl.ds(start, _WINA)],
                                 win_v.at[pl.ds(b * _WIN, _WINA)], wsem[b])

                @pl.when(jnp.logical_not(last))
                def _():
                    pltpu.async_copy(
                        tab_hbm.at[cg, c, pl.ds(start + _WINA, _WINB)],
                        win_v.at[pl.ds(b * _WIN + _WINA, _WINB)], wsem[b])

                @pl.when(last)
                def _():
                    pltpu.async_copy(tail_hbm.at[cg, c],
                                     win_v.at[pl.ds(b * _WIN + _WINA, 128)],
                                     wsem[b])

    def wait_win(f, b):
        last = f == _NUM_FIELDS - 1
        pltpu.make_async_copy(tab_hbm.at[0, 0, pl.ds(0, _WINA)],
                              win_v.at[pl.ds(b * _WIN, _WINA)],
                              wsem[b]).wait()

        @pl.when(jnp.logical_not(last))
        def _():
            pltpu.make_async_copy(tab_hbm.at[0, 0, pl.ds(0, _WINB)],
                                  win_v.at[pl.ds(b * _WIN + _WINA, _WINB)],
                                  wsem[b]).wait()

        @pl.when(last)
        def _():
            pltpu.make_async_copy(tab_hbm.at[0, 0, pl.ds(0, 128)],
                                  win_v.at[pl.ds(b * _WIN + _WINA, 128)],
                                  wsem[b]).wait()

    def fire_idx(f, b):
        pltpu.async_copy(xt_hbm.at[f], idx_v.at[b], isem[b])

    def wait_idx(b):
        pltpu.make_async_copy(xt_hbm.at[0], idx_v.at[b], isem[b]).wait()

    def wait_store(h):
        pltpu.make_async_copy(out_hbm.at[0, 0, pl.ds(0, 64), 0, :],
                              outb_v.at[h], osem[h]).wait()

    def fire_store(f, h):
        for c in range(8):
            @pl.when(ci == c)
            def _(c=c):
                pltpu.async_copy(outb_v.at[h],
                                 out_hbm.at[f, cg, pl.ds(64 * h, 64), c, :],
                                 osem[h])

    def process(f, b):
        wait_win(f, b)
        wait_idx(b)
        d = lax.rem(f * _FIELD_WIDTH, 128)
        for h in range(2):
            @pl.when(f >= 1)
            def _(h=h):
                wait_store(h)

            @plsc.parallel_loop(0, 64, step=1)
            def r_step(r, h=h):
                for j in range(8):
                    sl = pl.ds(j * _LANES, _LANES)
                    outb_v[h, r, sl] = plsc.load_gather(
                        win_v, [idx_v[b, 64 * h + r, sl] + (d + b * _WIN)])
            fire_store(f, h)

    fire_win(0, 0)
    fire_idx(0, 0)

    def g_step(g, carry):
        f0 = 2 * g
        fire_win(f0 + 1, 1)
        fire_idx(f0 + 1, 1)
        process(f0, 0)

        @pl.when(f0 + 2 < _NUM_FIELDS)
        def _():
            fire_win(f0 + 2, 0)
            fire_idx(f0 + 2, 0)

        process(f0 + 1, 1)
        return carry

    lax.fori_loop(0, _NUM_FIELDS // 2, g_step, 0)
    for h in range(2):
        wait_store(h)


@functools.partial(jax.jit, static_argnums=())
def kernel(x, table):
    xt = x.astype(jnp.int32).T.reshape(_NUM_FIELDS, _NG, 128)
    tab = table.T.reshape(4, 8, table.shape[0])
    tail = jnp.pad(
        lax.slice(table.T, (0, _TAIL_START), (_EMBED_DIM, table.shape[0])),
        ((0, 0), (0, 128 - (table.shape[0] - _TAIL_START))),
    ).reshape(4, 8, 128)
    mesh = plsc.VectorSubcoreMesh(core_axis_name="c", subcore_axis_name="s")
    run = pl.kernel(
        _body,
        out_type=jax.ShapeDtypeStruct((_NUM_FIELDS, 4, _NG, 8, 128),
                                      jnp.float32),
        mesh=mesh,
        scratch_types=[
            pltpu.VMEM((2 * _WIN,), jnp.float32),
            pltpu.VMEM((2, 128, 128), jnp.int32),
            pltpu.VMEM((2, 64, 128), jnp.float32),
        ] + [pltpu.SemaphoreType.DMA] * 6,
        compiler_params=pltpu.CompilerParams(needs_layout_passes=False),
    )
    out5 = run(xt, tab, tail)
    return out5.transpose(2, 4, 0, 1, 3).reshape(_BATCH, _NUM_FIELDS,
                                                 _EMBED_DIM)


# speedup vs baseline: 1.0326x; 1.0326x over previous
import functools

import jax
import jax.numpy as jnp
from jax import lax
from jax.experimental import pallas as pl
from jax.experimental.pallas import tpu as pltpu
from jax.experimental.pallas import tpu_sc as plsc

_NUM_FIELDS = 26
_FIELD_WIDTH = 38462
_EMBED_DIM = 32
_BATCH = 16384

_LANES = 16
_NW = 32
_NG = _BATCH // 128
_WINA = 38400
_WINB = 256
_WIN = _WINA + _WINB
_TAIL_START = 999936


def _body(xt_hbm, tab_hbm, tail_hbm, out_hbm, win_v, idx_v, outb_v, *sems):
    wsem = sems[0:2]
    isem = sems[2:4]
    osem = sems[4:6]
    wid = lax.axis_index("s") * 2 + lax.axis_index("c")
    cg = wid // 8
    ci = lax.rem(wid, 8)

    def fire_win(f, b):
        off = f * _FIELD_WIDTH
        d = lax.rem(off, 128)
        start = pl.multiple_of(off - d, 128)
        last = f == _NUM_FIELDS - 1
        for c in range(8):
            @pl.when(ci == c)
            def _(c=c):
                pltpu.async_copy(tab_hbm.at[cg, c, pl.ds(start, _WINA)],
                                 win_v.at[pl.ds(b * _WIN, _WINA)], wsem[b])

                @pl.when(jnp.logical_not(last))
                def _():
                    pltpu.async_copy(
                        tab_hbm.at[cg, c, pl.ds(start + _WINA, _WINB)],
                        win_v.at[pl.ds(b * _WIN + _WINA, _WINB)], wsem[b])

                @pl.when(last)
                def _():
                    pltpu.async_copy(tail_hbm.at[cg, c],
                                     win_v.at[pl.ds(b * _WIN + _WINA, 128)],
                                     wsem[b])

    def wait_win(f, b):
        last = f == _NUM_FIELDS - 1
        pltpu.make_async_copy(tab_hbm.at[0, 0, pl.ds(0, _WINA)],
                              win_v.at[pl.ds(b * _WIN, _WINA)],
                              wsem[b]).wait()

        @pl.when(jnp.logical_not(last))
        def _():
            pltpu.make_async_copy(tab_hbm.at[0, 0, pl.ds(0, _WINB)],
                                  win_v.at[pl.ds(b * _WIN + _WINA, _WINB)],
                                  wsem[b]).wait()

        @pl.when(last)
        def _():
            pltpu.make_async_copy(tab_hbm.at[0, 0, pl.ds(0, 128)],
                                  win_v.at[pl.ds(b * _WIN + _WINA, 128)],
                                  wsem[b]).wait()

    def fire_idx(f, b):
        for ff in range(_NUM_FIELDS):
            @pl.when(f == ff)
            def _(ff=ff):
                pltpu.async_copy(xt_hbm.at[ff],
                                 idx_v.at[pl.ds(b * _BATCH, _BATCH)], isem[b])

    def wait_idx(b):
        pltpu.make_async_copy(xt_hbm.at[0],
                              idx_v.at[pl.ds(b * _BATCH, _BATCH)],
                              isem[b]).wait()

    def wait_store(h):
        pltpu.make_async_copy(out_hbm.at[0, 0, pl.ds(0, 64), 0, :],
                              outb_v.at[h], osem[h]).wait()

    def fire_store(f, h):
        for c in range(8):
            @pl.when(ci == c)
            def _(c=c):
                pltpu.async_copy(outb_v.at[h],
                                 out_hbm.at[f, cg, pl.ds(64 * h, 64), c, :],
                                 osem[h])

    def process(f, b):
        wait_win(f, b)
        wait_idx(b)
        d = lax.rem(f * _FIELD_WIDTH, 128)
        for h in range(2):
            @pl.when(f >= 1)
            def _(h=h):
                wait_store(h)

            @plsc.parallel_loop(0, 64, step=1)
            def r_step(r, h=h):
                for j in range(8):
                    sl = pl.ds(j * _LANES, _LANES)
                    base = b * _BATCH + (64 * h + r) * 128 + j * _LANES
                    outb_v[h, r, sl] = plsc.load_gather(
                        win_v, [idx_v[pl.ds(base, _LANES)] + (d + b * _WIN)])
            fire_store(f, h)

    fire_win(0, 0)
    fire_idx(0, 0)

    def g_step(g, carry):
        f0 = 2 * g
        fire_win(f0 + 1, 1)
        fire_idx(f0 + 1, 1)
        process(f0, 0)

        @pl.when(f0 + 2 < _NUM_FIELDS)
        def _():
            fire_win(f0 + 2, 0)
            fire_idx(f0 + 2, 0)

        process(f0 + 1, 1)
        return carry

    lax.fori_loop(0, _NUM_FIELDS // 2, g_step, 0)
    for h in range(2):
        wait_store(h)


@functools.partial(jax.jit, static_argnums=())
def kernel(x, table):
    xt = x.astype(jnp.int32).T
    tab = table.T.reshape(4, 8, table.shape[0])
    tail = jnp.pad(
        lax.slice(table.T, (0, _TAIL_START), (_EMBED_DIM, table.shape[0])),
        ((0, 0), (0, 128 - (table.shape[0] - _TAIL_START))),
    ).reshape(4, 8, 128)
    mesh = plsc.VectorSubcoreMesh(core_axis_name="c", subcore_axis_name="s")
    run = pl.kernel(
        _body,
        out_type=jax.ShapeDtypeStruct((_NUM_FIELDS, 4, _NG, 8, 128),
                                      jnp.float32),
        mesh=mesh,
        scratch_types=[
            pltpu.VMEM((2 * _WIN,), jnp.float32),
            pltpu.VMEM((2 * _BATCH,), jnp.int32),
            pltpu.VMEM((2, 64, 128), jnp.float32),
        ] + [pltpu.SemaphoreType.DMA] * 6,
        compiler_params=pltpu.CompilerParams(needs_layout_passes=False),
    )
    out5 = run(xt, tab, tail)
    return out5.transpose(2, 4, 0, 1, 3).reshape(_BATCH, _NUM_FIELDS,
                                                 _EMBED_DIM)
